# Initial kernel scaffold; baseline (speedup 1.0000x reference)
import jax
import jax.numpy as jnp
from jax.experimental import pallas as pl
from jax.experimental.pallas import tpu as pltpu

TEMP = 0.07
N_CLASSES = 65536
D = 128
B = 1024
N_CAMS = 8

T = 2048
NT = N_CLASSES // T


def _main_body(feat_ref, mem_ref, ccam_ref, lab_ref, cams_ref, out_ref,
               acc_ref, s_ref, cc_ref):
    t = pl.program_id(0)

    @pl.when(t == 0)
    def _init():
        acc_ref[...] = jnp.zeros_like(acc_ref)
        s_ref[...] = jnp.zeros_like(s_ref)
        cc_ref[...] = jnp.zeros_like(cc_ref)

    scores = jax.lax.dot_general(
        feat_ref[...], mem_ref[...],
        (((1,), (1,)), ((), ())),
        preferred_element_type=jnp.float32) * (1.0 / TEMP)

    cam_row = ccam_ref[0]
    oh = (jax.lax.broadcasted_iota(jnp.int32, (128, T), 0)
          == cam_row).astype(jnp.float32)
    e = jnp.exp(scores)
    acc_ref[...] += jax.lax.dot_general(
        e, oh, (((1,), (1,)), ((), ())),
        preferred_element_type=jnp.float32)

    col = jax.lax.broadcasted_iota(jnp.int32, (B, T), 1) + t * T
    hit = col == lab_ref[...]
    s_ref[...] += jnp.sum(jnp.where(hit, scores, 0.0), axis=1, keepdims=True)
    cc_ref[...] += jnp.sum(
        jnp.where(hit, cam_row.astype(jnp.float32), 0.0), axis=1,
        keepdims=True)

    @pl.when(t == NT - 1)
    def _finish():
        acc = acc_ref[...]
        cams_f = cams_ref[...].astype(jnp.float32)
        cam_oh = (jax.lax.broadcasted_iota(jnp.int32, (B, 128), 1)
                  == cams_ref[...]).astype(jnp.float32)
        counts = jnp.sum(cam_oh, axis=0, keepdims=True)
        inv = 1.0 / jnp.maximum(counts, 1.0)
        lse = jnp.log(jnp.where(acc > 0.0, acc, 1.0))
        lse_sel = jnp.sum(cam_oh * lse, axis=1, keepdims=True)
        inv_sel = jnp.sum(cam_oh * inv, axis=1, keepdims=True)
        match = cc_ref[...] == cams_f
        per_row = jnp.where(match, (lse_sel - s_ref[...]) * inv_sel, 0.0)
        out_ref[0, 0] = jnp.sum(per_row)


def kernel(features, labels, cams, epoch, class_memory, class_camera):
    del epoch
    ccam3 = class_camera.reshape(NT, 1, T).astype(jnp.int32)
    lab2 = labels.reshape(B, 1).astype(jnp.int32)
    cams2 = cams.reshape(B, 1).astype(jnp.int32)

    loss = pl.pallas_call(
        _main_body,
        grid=(NT,),
        in_specs=[
            pl.BlockSpec((B, D), lambda t: (0, 0)),
            pl.BlockSpec((T, D), lambda t: (t, 0)),
            pl.BlockSpec((1, 1, T), lambda t: (t, 0, 0)),
            pl.BlockSpec((B, 1), lambda t: (0, 0)),
            pl.BlockSpec((B, 1), lambda t: (0, 0)),
        ],
        out_specs=pl.BlockSpec((1, 1), lambda t: (0, 0)),
        out_shape=jax.ShapeDtypeStruct((1, 1), jnp.float32),
        scratch_shapes=[
            pltpu.VMEM((B, 128), jnp.float32),
            pltpu.VMEM((B, 1), jnp.float32),
            pltpu.VMEM((B, 1), jnp.float32),
        ],
    )(features, class_memory, ccam3, lab2, cams2)
    return loss.reshape(())


# speedup vs baseline: 16.9081x; 16.9081x over previous
import functools

import jax
import jax.numpy as jnp
from jax import lax
from jax.experimental import pallas as pl
from jax.experimental.pallas import tpu as pltpu
from jax.experimental.pallas import tpu_sc as plsc

TEMP = 0.07
N_CLASSES = 65536
D = 128
B = 1024
N_CAMS = 8

T = 2048
NT = N_CLASSES // T

_SC_INFO = plsc.get_sparse_core_info()
_NW = _SC_INFO.num_cores * _SC_INFO.num_subcores
_RPW = B // _NW


def _sc_body(mem_hbm, lab_hbm, ccam_hbm,
             rows_hbm, cclab_hbm,
             idx_v, rows_v, cc_v, sem_a, sem_b):
    wid = lax.axis_index("s") * _SC_INFO.num_cores + lax.axis_index("c")
    base = wid * _RPW
    pltpu.sync_copy(lab_hbm.at[pl.ds(base, _RPW)], idx_v)
    ca = pltpu.async_copy(mem_hbm.at[idx_v], rows_v, sem_a)
    cb = pltpu.async_copy(ccam_hbm.at[idx_v], cc_v, sem_b)
    ca.wait()
    cb.wait()
    pltpu.sync_copy(rows_v, rows_hbm.at[pl.ds(base, _RPW)])
    pltpu.sync_copy(cc_v, cclab_hbm.at[pl.ds(base, _RPW)])


_sc_lookup = functools.partial(
    pl.kernel, _sc_body,
    mesh=plsc.VectorSubcoreMesh(core_axis_name="c", subcore_axis_name="s"),
    out_type=[jax.ShapeDtypeStruct((B, D), jnp.float32),
              jax.ShapeDtypeStruct((B,), jnp.int32)],
    scratch_types=[
        pltpu.VMEM((_RPW,), jnp.int32),
        pltpu.VMEM((_RPW, D), jnp.float32),
        pltpu.VMEM((_RPW,), jnp.int32),
        pltpu.SemaphoreType.DMA,
        pltpu.SemaphoreType.DMA,
    ],
)


def _main_body(feat_ref, mem_ref, ccam_ref, acc_ref):
    t = pl.program_id(0)
    scores = lax.dot_general(
        feat_ref[...].astype(jnp.bfloat16), mem_ref[...].astype(jnp.bfloat16),
        (((1,), (1,)), ((), ())),
        preferred_element_type=jnp.float32) * (1.0 / TEMP)
    oh = (lax.broadcasted_iota(jnp.int32, (128, T), 0)
          == ccam_ref[0]).astype(jnp.bfloat16)
    part = lax.dot_general(
        jnp.exp(scores).astype(jnp.bfloat16), oh, (((1,), (1,)), ((), ())),
        preferred_element_type=jnp.float32)

    @pl.when(t == 0)
    def _init():
        acc_ref[...] = part

    @pl.when(t != 0)
    def _accum():
        acc_ref[...] += part


def _fin_body(acc_ref, feat_ref, rows_ref, cclab_ref, cams_ref, out_ref):
    acc = acc_ref[...]
    s_col = jnp.sum(feat_ref[...] * rows_ref[...], axis=1,
                    keepdims=True) * (1.0 / TEMP)
    cam_oh = (lax.broadcasted_iota(jnp.int32, (B, 128), 1)
              == cams_ref[...]).astype(jnp.float32)
    counts = jnp.sum(cam_oh, axis=0, keepdims=True)
    inv = 1.0 / jnp.maximum(counts, 1.0)
    lse = jnp.log(jnp.where(acc > 0.0, acc, 1.0))
    lse_sel = jnp.sum(cam_oh * lse, axis=1, keepdims=True)
    inv_sel = jnp.sum(cam_oh * inv, axis=1, keepdims=True)
    per_row = jnp.where(cclab_ref[...] == cams_ref[...],
                        (lse_sel - s_col) * inv_sel, 0.0)
    out_ref[...] = jnp.sum(per_row, axis=0, keepdims=True)


def kernel(features, labels, cams, epoch, class_memory, class_camera):
    del epoch
    lab1 = labels.astype(jnp.int32)
    cams1 = cams.astype(jnp.int32)
    ccam1 = class_camera.astype(jnp.int32)

    rows, cclab = _sc_lookup()(class_memory, lab1, ccam1)

    acc = pl.pallas_call(
        _main_body,
        grid=(NT,),
        in_specs=[
            pl.BlockSpec((B, D), lambda t: (0, 0)),
            pl.BlockSpec((T, D), lambda t: (t, 0)),
            pl.BlockSpec((1, 1, T), lambda t: (t, 0, 0)),
        ],
        out_specs=pl.BlockSpec((B, 128), lambda t: (0, 0)),
        out_shape=jax.ShapeDtypeStruct((B, 128), jnp.float32),
    )(features, class_memory, ccam1.reshape(NT, 1, T))

    loss = pl.pallas_call(
        _fin_body,
        in_specs=[
            pl.BlockSpec((B, 128), lambda: (0, 0)),
            pl.BlockSpec((B, D), lambda: (0, 0)),
            pl.BlockSpec((B, D), lambda: (0, 0)),
            pl.BlockSpec((B, 1), lambda: (0, 0)),
            pl.BlockSpec((B, 1), lambda: (0, 0)),
        ],
        out_specs=pl.BlockSpec((1, 1), lambda: (0, 0)),
        out_shape=jax.ShapeDtypeStruct((1, 1), jnp.float32),
    )(acc, features, rows, cclab.reshape(B, 1), cams1.reshape(B, 1))
    return loss.reshape(())


# speedup vs baseline: 16.9625x; 1.0032x over previous
import functools

import jax
import jax.numpy as jnp
from jax import lax
from jax.experimental import pallas as pl
from jax.experimental.pallas import tpu as pltpu
from jax.experimental.pallas import tpu_sc as plsc

TEMP = 0.07
N_CLASSES = 65536
D = 128
B = 1024
N_CAMS = 8

T = 2048
NT = N_CLASSES // T

_SC_INFO = plsc.get_sparse_core_info()
_NW = _SC_INFO.num_cores * _SC_INFO.num_subcores
_RPW = B // _NW


def _sc_body(mem_hbm, lab_hbm, ccam_hbm,
             rows_hbm, cclab_hbm,
             idx_v, rows_v, cc_v, sem_a, sem_b):
    wid = lax.axis_index("s") * _SC_INFO.num_cores + lax.axis_index("c")
    base = wid * _RPW
    pltpu.sync_copy(lab_hbm.at[pl.ds(base, _RPW)], idx_v)
    ca = pltpu.async_copy(mem_hbm.at[idx_v], rows_v, sem_a)
    cb = pltpu.async_copy(ccam_hbm.at[idx_v], cc_v, sem_b)
    ca.wait()
    cb.wait()
    pltpu.sync_copy(rows_v, rows_hbm.at[pl.ds(base, _RPW)])
    pltpu.sync_copy(cc_v, cclab_hbm.at[pl.ds(base, _RPW)])


_sc_lookup = functools.partial(
    pl.kernel, _sc_body,
    mesh=plsc.VectorSubcoreMesh(core_axis_name="c", subcore_axis_name="s"),
    out_type=[jax.ShapeDtypeStruct((B, D), jnp.float32),
              jax.ShapeDtypeStruct((B,), jnp.int32)],
    scratch_types=[
        pltpu.VMEM((_RPW,), jnp.int32),
        pltpu.VMEM((_RPW, D), jnp.float32),
        pltpu.VMEM((_RPW,), jnp.int32),
        pltpu.SemaphoreType.DMA,
        pltpu.SemaphoreType.DMA,
    ],
)


_LOG2E = 1.4426950408889634


def _main_body(feat_ref, mem_ref, ccam_ref, acc_ref):
    t = pl.program_id(0)
    feat = (feat_ref[...] * (_LOG2E / TEMP)).astype(jnp.bfloat16)
    scores2 = lax.dot_general(
        feat, mem_ref[...].astype(jnp.bfloat16),
        (((1,), (1,)), ((), ())),
        preferred_element_type=jnp.float32).astype(jnp.bfloat16)
    oh = (lax.broadcasted_iota(jnp.int32, (128, T), 0)
          == ccam_ref[0]).astype(jnp.bfloat16)
    part = lax.dot_general(
        jnp.exp2(scores2), oh, (((1,), (1,)), ((), ())),
        preferred_element_type=jnp.float32)

    @pl.when(t == 0)
    def _init():
        acc_ref[...] = part

    @pl.when(t != 0)
    def _accum():
        acc_ref[...] += part


def _fin_body(acc_ref, feat_ref, rows_ref, cclab_ref, cams_ref, out_ref):
    acc = acc_ref[...]
    s_col = jnp.sum(feat_ref[...] * rows_ref[...], axis=1,
                    keepdims=True) * (1.0 / TEMP)
    cam_oh = (lax.broadcasted_iota(jnp.int32, (B, 128), 1)
              == cams_ref[...]).astype(jnp.float32)
    counts = jnp.sum(cam_oh, axis=0, keepdims=True)
    inv = 1.0 / jnp.maximum(counts, 1.0)
    lse = jnp.log2(jnp.where(acc > 0.0, acc, 1.0)) * 0.6931471805599453
    lse_sel = jnp.sum(cam_oh * lse, axis=1, keepdims=True)
    inv_sel = jnp.sum(cam_oh * inv, axis=1, keepdims=True)
    per_row = jnp.where(cclab_ref[...] == cams_ref[...],
                        (lse_sel - s_col) * inv_sel, 0.0)
    out_ref[...] = jnp.sum(per_row, axis=0, keepdims=True)


def kernel(features, labels, cams, epoch, class_memory, class_camera):
    del epoch
    lab1 = labels.astype(jnp.int32)
    cams1 = cams.astype(jnp.int32)
    ccam1 = class_camera.astype(jnp.int32)

    rows, cclab = _sc_lookup()(class_memory, lab1, ccam1)

    acc = pl.pallas_call(
        _main_body,
        grid=(NT,),
        in_specs=[
            pl.BlockSpec((B, D), lambda t: (0, 0)),
            pl.BlockSpec((T, D), lambda t: (t, 0)),
            pl.BlockSpec((1, 1, T), lambda t: (t, 0, 0)),
        ],
        out_specs=pl.BlockSpec((B, 128), lambda t: (0, 0)),
        out_shape=jax.ShapeDtypeStruct((B, 128), jnp.float32),
    )(features, class_memory, ccam1.reshape(NT, 1, T))

    loss = pl.pallas_call(
        _fin_body,
        in_specs=[
            pl.BlockSpec((B, 128), lambda: (0, 0)),
            pl.BlockSpec((B, D), lambda: (0, 0)),
            pl.BlockSpec((B, D), lambda: (0, 0)),
            pl.BlockSpec((B, 1), lambda: (0, 0)),
            pl.BlockSpec((B, 1), lambda: (0, 0)),
        ],
        out_specs=pl.BlockSpec((1, 1), lambda: (0, 0)),
        out_shape=jax.ShapeDtypeStruct((1, 1), jnp.float32),
    )(acc, features, rows, cclab.reshape(B, 1), cams1.reshape(B, 1))
    return loss.reshape(())
